# baseline (speedup 1.0000x reference)
import functools

import jax
import jax.numpy as jnp
from jax import lax
from jax.experimental import pallas as pl
from jax.experimental.pallas import tpu as pltpu
from jax.experimental.pallas import tpu_sc as plsc

N = 10000
E = 160000
D_IN = 256
H1 = 256
H2 = 128

NP = 10240
NC = 2
NS = 16
NW = NC * NS
EPAD = 163840
EPW = EPAD // NW
EB = 64
STRIPE = NP // NS


def _sc_mesh():
    return plsc.VectorSubcoreMesh(
        core_axis_name="c", subcore_axis_name="s",
        num_cores=NC, num_subcores=NS)


def _deg_body(ei_hbm, out_hbm, idx_v, deg_v, ones16):
    c = lax.axis_index("c")
    s = lax.axis_index("s")
    wid = s * NC + c
    base = wid * EPW

    zeros16 = jnp.zeros((16,), jnp.float32)

    def zero_body(i, _):
        deg_v[pl.ds(i * 16, 16)] = zeros16
        return 0

    lax.fori_loop(0, NP // 16, zero_body, 0)

    pltpu.sync_copy(ei_hbm.at[pl.ds(base, EPW)], idx_v.at[0])
    pltpu.sync_copy(ei_hbm.at[pl.ds(EPAD + base, EPW)], idx_v.at[1])

    def acc_body(i, _):
        src16 = idx_v[0, pl.ds(i * 16, 16)]
        dst16 = idx_v[1, pl.ds(i * 16, 16)]
        plsc.addupdate_scatter(deg_v, [src16], ones16)
        plsc.addupdate_scatter(deg_v, [dst16], ones16)
        return 0

    lax.fori_loop(0, EPW // 16, acc_body, 0)

    pltpu.sync_copy(deg_v, out_hbm.at[wid])


def _deg_partials(ei_flat):
    kfn = functools.partial(
        pl.kernel,
        out_type=jax.ShapeDtypeStruct((NW, NP), jnp.float32),
        mesh=_sc_mesh(),
        compiler_params=pltpu.CompilerParams(needs_layout_passes=False),
        scratch_types=[
            pltpu.VMEM((2, EPW), jnp.int32),
            pltpu.VMEM((NP,), jnp.float32),
        ],
    )(lambda ei, out, idx_v, deg_v: _deg_body(
        ei, out, idx_v, deg_v, jnp.ones((16,), jnp.float32)))
    return kfn(ei_flat)


def _norm_body(deg_ref, out_ref):
    deg = jnp.sum(deg_ref[...], axis=0)
    out_ref[...] = lax.rsqrt(jnp.maximum(deg, 1.0))


def _norm(deg_partials):
    return pl.pallas_call(
        _norm_body,
        out_shape=jax.ShapeDtypeStruct((NP,), jnp.float32),
    )(deg_partials)


def _matmul1_body(x_ref, w_ref, n_ref, out_ref):
    hw = jnp.dot(x_ref[...], w_ref[...], preferred_element_type=jnp.float32)
    out_ref[0, :, :] = hw * n_ref[...][:, None]


def _scaled_matmul_split(h, W, norm, bm):
    K = h.shape[1]
    Dh = W.shape[1] // 2
    grid = (2, NP // bm)
    return pl.pallas_call(
        _matmul1_body,
        grid=grid,
        in_specs=[
            pl.BlockSpec((bm, K), lambda c, i: (i, 0)),
            pl.BlockSpec((K, Dh), lambda c, i: (0, c)),
            pl.BlockSpec((bm,), lambda c, i: (i,)),
        ],
        out_specs=pl.BlockSpec((1, bm, Dh), lambda c, i: (c, i, 0)),
        out_shape=jax.ShapeDtypeStruct((2, NP, Dh), jnp.float32),
    )(h, W, norm).reshape(2 * NP, Dh)


NBUF = 3


def _msg_body(table_hbm, ei_hbm, out_hbm, src_v, dst_v, rows_v,
              agg_sh, gsem, ssem, xsem, split_cols, ept):
    c = lax.axis_index("c")
    s = lax.axis_index("s")
    nsteps = ept // EB
    if split_cols:
        row_off = c * NP
        ebase = s * ept
    else:
        row_off = 0
        ebase = (c * NS + s) * ept

    src_desc = pltpu.async_copy(ei_hbm.at[pl.ds(ebase, ept)], src_v, xsem)
    dst_desc = pltpu.async_copy(
        ei_hbm.at[pl.ds(EPAD + ebase, ept)], dst_v, xsem)

    zeros16 = jnp.zeros((16,), jnp.float32)

    def zero_body(r, _):
        for j in range(128 // 16):
            rows_v[0, r, pl.ds(j * 16, 16)] = zeros16
        return 0

    lax.fori_loop(0, EB, zero_body, 0)
    for k in range(STRIPE // EB):
        pltpu.sync_copy(rows_v.at[0],
                        agg_sh.at[pl.ds(s * STRIPE + k * EB, EB)])

    src_desc.wait()
    dst_desc.wait()
    if split_cols:
        def adj_body(i, _):
            src_v[pl.ds(i * 16, 16)] = src_v[pl.ds(i * 16, 16)] + row_off
            return 0

        lax.fori_loop(0, ept // 16, adj_body, 0)
    plsc.subcore_barrier()

    def issue_gather(g, b):
        pltpu.async_copy(table_hbm.at[src_v.at[pl.ds(g * EB, EB)]],
                         rows_v.at[b], gsem.at[b])

    def wait_gather(b):
        pltpu.make_async_copy(table_hbm.at[pl.ds(0, EB)], rows_v.at[b],
                              gsem.at[b]).wait()

    def issue_scatter(g, b):
        pltpu.async_copy(rows_v.at[b],
                         agg_sh.at[dst_v.at[pl.ds(g * EB, EB)]],
                         ssem.at[b], add=True)

    def wait_scatter(b):
        pltpu.make_async_copy(rows_v.at[b], agg_sh.at[pl.ds(0, EB)],
                              ssem.at[b]).wait()

    def step(g, b, issue):
        wait_gather(b)
        issue_scatter(g, b)
        b2 = (b + 2) % NBUF
        wait_scatter(b2)
        if issue:
            issue_gather(g + 2, b2)

    issue_gather(0, 0)
    issue_gather(1, 1)
    wait_gather(0)
    issue_scatter(0, 0)
    issue_gather(2, 2)
    step(1, 1, True)

    tail0 = 2 + 3 * ((nsteps - 4) // 3)

    def steady(grp, _):
        for j in range(3):
            g = 2 + grp * 3 + j
            step(g, (2 + j) % NBUF, True)
        return 0

    lax.fori_loop(0, (tail0 - 2) // 3, steady, 0)

    for gg in range(tail0, nsteps):
        step(gg, gg % NBUF, gg + 2 < nsteps)
    wait_scatter((nsteps - 1) % NBUF)
    plsc.subcore_barrier()

    for k in range(STRIPE // EB):
        r0 = s * STRIPE + k * EB
        pltpu.sync_copy(agg_sh.at[pl.ds(r0, EB)], rows_v.at[0])
        pltpu.sync_copy(rows_v.at[0], out_hbm.at[c, pl.ds(r0, EB)])


def _message_pass(table, ei_flat, split_cols):
    ept = (EPAD // NS) if split_cols else EPW
    kfn = functools.partial(
        pl.kernel,
        out_type=jax.ShapeDtypeStruct((2, NP, 128), jnp.float32),
        mesh=_sc_mesh(),
        compiler_params=pltpu.CompilerParams(needs_layout_passes=False),
        scratch_types=[
            pltpu.VMEM((ept,), jnp.int32),
            pltpu.VMEM((ept,), jnp.int32),
            pltpu.VMEM((NBUF, EB, 128), jnp.float32),
            pltpu.VMEM_SHARED((NP, 128), jnp.float32),
            pltpu.SemaphoreType.DMA((NBUF,)),
            pltpu.SemaphoreType.DMA((NBUF,)),
            pltpu.SemaphoreType.DMA,
        ],
    )(functools.partial(_msg_body, split_cols=split_cols, ept=ept))
    return kfn(table, ei_flat)


def _layer2_body(agg_ref, n_ref, w_ref, out_ref):
    nvec = n_ref[...][:, None]
    h1a = jnp.maximum(agg_ref[0, :, :] * nvec, 0.0)
    h1b = jnp.maximum(agg_ref[1, :, :] * nvec, 0.0)
    hw = (jnp.dot(h1a, w_ref[0, :, :], preferred_element_type=jnp.float32)
          + jnp.dot(h1b, w_ref[1, :, :], preferred_element_type=jnp.float32))
    out_ref[...] = hw * nvec


def _layer2(agg1, norm, W2, bm):
    W2r = W2.reshape(2, H1 // 2, H2)
    grid = (NP // bm,)
    return pl.pallas_call(
        _layer2_body,
        grid=grid,
        in_specs=[
            pl.BlockSpec((2, bm, H1 // 2), lambda i: (0, i, 0)),
            pl.BlockSpec((bm,), lambda i: (i,)),
            pl.BlockSpec((2, H1 // 2, H2), lambda i: (0, 0, 0)),
        ],
        out_specs=pl.BlockSpec((bm, H2), lambda i: (i, 0)),
        out_shape=jax.ShapeDtypeStruct((NP, H2), jnp.float32),
    )(agg1, norm, W2r)


def _zfuse_body(agg_ref, n_ref, out_ref):
    z = (agg_ref[0, :, :] + agg_ref[1, :, :]) * n_ref[...][:, None]
    out_ref[...] = z.astype(jnp.bfloat16)


def _zfuse(agg2, norm):
    bm = 2048
    return pl.pallas_call(
        _zfuse_body,
        grid=(NP // bm,),
        in_specs=[
            pl.BlockSpec((2, bm, H2), lambda i: (0, i, 0)),
            pl.BlockSpec((bm,), lambda i: (i,)),
        ],
        out_specs=pl.BlockSpec((bm, H2), lambda i: (i, 0)),
        out_shape=jax.ShapeDtypeStruct((NP, H2), jnp.bfloat16),
    )(agg2, norm)


def _decoder_body(a_ref, b_ref, out_ref):
    res = lax.dot_general(
        a_ref[...], b_ref[...], (((1,), (1,)), ((), ())),
        preferred_element_type=jnp.float32)
    out_ref[...] = res.astype(jnp.bfloat16)


def _decoder(z, bm, bn):
    grid = (pl.cdiv(N, bm), pl.cdiv(N, bn))
    out = pl.pallas_call(
        _decoder_body,
        grid=grid,
        in_specs=[
            pl.BlockSpec((bm, H2), lambda i, j: (i, 0)),
            pl.BlockSpec((bn, H2), lambda i, j: (j, 0)),
        ],
        out_specs=pl.BlockSpec((bm, bn), lambda i, j: (i, j)),
        out_shape=jax.ShapeDtypeStruct((N, N), jnp.bfloat16),
    )(z, z)
    return jnp.reshape(out, (-1,)).astype(jnp.float32)


def kernel(x, edge_index, W1, W2):
    xp = jnp.pad(x, ((0, NP - N), (0, 0)))
    sent = N + (jnp.arange(EPAD, dtype=jnp.int32) % (NP - N))
    src_p = sent.at[:E].set(edge_index[0])
    dst_p = sent.at[:E].set(edge_index[1])
    ei_flat = jnp.concatenate([src_p, dst_p])

    deg_p = _deg_partials(ei_flat)
    norm = _norm(deg_p)

    hwn1 = _scaled_matmul_split(xp, W1, norm, bm=1024)
    agg1 = _message_pass(hwn1, ei_flat, split_cols=True)

    hwn2 = _layer2(agg1, norm, W2, bm=1024)
    agg2 = _message_pass(hwn2, ei_flat, split_cols=False)

    z = _zfuse(agg2, norm)
    return _decoder(z, bm=2048, bn=2048)


# speedup vs baseline: 1.0054x; 1.0054x over previous
import functools

import jax
import jax.numpy as jnp
import numpy as np
from jax import lax
from jax.experimental import pallas as pl
from jax.experimental.pallas import tpu as pltpu
from jax.experimental.pallas import tpu_sc as plsc

N = 10000
E = 160000
D_IN = 256
H1 = 256
H2 = 128

NP = 10240
NC = 2
NS = 16
NW = NC * NS
EPAD = 163840
EPW = EPAD // NW
EB = 64
STRIPE = NP // NS


def _sc_mesh():
    return plsc.VectorSubcoreMesh(
        core_axis_name="c", subcore_axis_name="s",
        num_cores=NC, num_subcores=NS)


def _deg_body(ei_hbm, out_hbm, idx_v, deg_v, ones16):
    c = lax.axis_index("c")
    s = lax.axis_index("s")
    wid = s * NC + c
    base = wid * EPW

    zeros16 = jnp.zeros((16,), jnp.float32)

    def zero_body(i, _):
        deg_v[pl.ds(i * 16, 16)] = zeros16
        return 0

    lax.fori_loop(0, NP // 16, zero_body, 0)

    pltpu.sync_copy(ei_hbm.at[pl.ds(base, EPW)], idx_v.at[0])
    pltpu.sync_copy(ei_hbm.at[pl.ds(EPAD + base, EPW)], idx_v.at[1])

    def acc_body(i, _):
        src16 = idx_v[0, pl.ds(i * 16, 16)]
        dst16 = idx_v[1, pl.ds(i * 16, 16)]
        plsc.addupdate_scatter(deg_v, [src16], ones16)
        plsc.addupdate_scatter(deg_v, [dst16], ones16)
        return 0

    lax.fori_loop(0, EPW // 16, acc_body, 0)

    pltpu.sync_copy(deg_v, out_hbm.at[wid])


def _deg_partials(ei_flat):
    kfn = functools.partial(
        pl.kernel,
        out_type=jax.ShapeDtypeStruct((NW, NP), jnp.float32),
        mesh=_sc_mesh(),
        compiler_params=pltpu.CompilerParams(needs_layout_passes=False),
        scratch_types=[
            pltpu.VMEM((2, EPW), jnp.int32),
            pltpu.VMEM((NP,), jnp.float32),
        ],
    )(lambda ei, out, idx_v, deg_v: _deg_body(
        ei, out, idx_v, deg_v, jnp.ones((16,), jnp.float32)))
    return kfn(ei_flat)


def _norm_body(deg_ref, out_ref):
    deg = jnp.sum(deg_ref[...], axis=0)
    out_ref[...] = lax.rsqrt(jnp.maximum(deg, 1.0))


def _norm(deg_partials):
    return pl.pallas_call(
        _norm_body,
        out_shape=jax.ShapeDtypeStruct((NP,), jnp.float32),
    )(deg_partials)


def _matmul1_body(x_ref, w_ref, n_ref, out_ref):
    hw = jnp.dot(x_ref[...], w_ref[...], preferred_element_type=jnp.float32)
    out_ref[0, :, :] = hw * n_ref[...][:, None]


def _scaled_matmul_split(h, W, norm, bm):
    K = h.shape[1]
    Dh = W.shape[1] // 2
    grid = (2, NP // bm)
    return pl.pallas_call(
        _matmul1_body,
        grid=grid,
        in_specs=[
            pl.BlockSpec((bm, K), lambda c, i: (i, 0)),
            pl.BlockSpec((K, Dh), lambda c, i: (0, c)),
            pl.BlockSpec((bm,), lambda c, i: (i,)),
        ],
        out_specs=pl.BlockSpec((1, bm, Dh), lambda c, i: (c, i, 0)),
        out_shape=jax.ShapeDtypeStruct((2, NP, Dh), jnp.float32),
    )(h, W, norm).reshape(2 * NP, Dh)


NBUF = 3


def _msg_body(table_hbm, ei_hbm, out_hbm, src_v, dst_v, rows_v,
              agg_sh, gsem, ssem, xsem, split_cols, ept):
    c = lax.axis_index("c")
    s = lax.axis_index("s")
    nsteps = ept // EB
    if split_cols:
        row_off = c * NP
        ebase = s * ept
    else:
        row_off = 0
        ebase = (c * NS + s) * ept

    src_desc = pltpu.async_copy(ei_hbm.at[pl.ds(ebase, ept)], src_v, xsem)
    dst_desc = pltpu.async_copy(
        ei_hbm.at[pl.ds(EPAD + ebase, ept)], dst_v, xsem)

    zeros16 = jnp.zeros((16,), jnp.float32)

    def zero_body(r, _):
        for j in range(128 // 16):
            rows_v[0, r, pl.ds(j * 16, 16)] = zeros16
        return 0

    lax.fori_loop(0, EB, zero_body, 0)
    for k in range(STRIPE // EB):
        pltpu.sync_copy(rows_v.at[0],
                        agg_sh.at[pl.ds(s * STRIPE + k * EB, EB)])

    src_desc.wait()
    dst_desc.wait()
    if split_cols:
        def adj_body(i, _):
            src_v[pl.ds(i * 16, 16)] = src_v[pl.ds(i * 16, 16)] + row_off
            return 0

        lax.fori_loop(0, ept // 16, adj_body, 0)
    plsc.subcore_barrier()

    def issue_gather(g, b):
        pltpu.async_copy(table_hbm.at[src_v.at[pl.ds(g * EB, EB)]],
                         rows_v.at[b], gsem.at[b])

    def wait_gather(b):
        pltpu.make_async_copy(table_hbm.at[pl.ds(0, EB)], rows_v.at[b],
                              gsem.at[b]).wait()

    def issue_scatter(g, b):
        pltpu.async_copy(rows_v.at[b],
                         agg_sh.at[dst_v.at[pl.ds(g * EB, EB)]],
                         ssem.at[b], add=True)

    def wait_scatter(b):
        pltpu.make_async_copy(rows_v.at[b], agg_sh.at[pl.ds(0, EB)],
                              ssem.at[b]).wait()

    def step(g, b, issue):
        wait_gather(b)
        issue_scatter(g, b)
        b2 = (b + 2) % NBUF
        wait_scatter(b2)
        if issue:
            issue_gather(g + 2, b2)

    issue_gather(0, 0)
    issue_gather(1, 1)
    wait_gather(0)
    issue_scatter(0, 0)
    issue_gather(2, 2)
    step(1, 1, True)

    tail0 = 2 + 3 * ((nsteps - 4) // 3)

    def steady(grp, _):
        for j in range(3):
            g = 2 + grp * 3 + j
            step(g, (2 + j) % NBUF, True)
        return 0

    lax.fori_loop(0, (tail0 - 2) // 3, steady, 0)

    for gg in range(tail0, nsteps):
        step(gg, gg % NBUF, gg + 2 < nsteps)
    wait_scatter((nsteps - 1) % NBUF)
    plsc.subcore_barrier()

    for k in range(STRIPE // EB):
        r0 = s * STRIPE + k * EB
        pltpu.sync_copy(agg_sh.at[pl.ds(r0, EB)], rows_v.at[0])
        pltpu.sync_copy(rows_v.at[0], out_hbm.at[c, pl.ds(r0, EB)])


def _message_pass(table, ei_flat, split_cols):
    ept = (EPAD // NS) if split_cols else EPW
    kfn = functools.partial(
        pl.kernel,
        out_type=jax.ShapeDtypeStruct((2, NP, 128), jnp.float32),
        mesh=_sc_mesh(),
        compiler_params=pltpu.CompilerParams(needs_layout_passes=False),
        scratch_types=[
            pltpu.VMEM((ept,), jnp.int32),
            pltpu.VMEM((ept,), jnp.int32),
            pltpu.VMEM((NBUF, EB, 128), jnp.float32),
            pltpu.VMEM_SHARED((NP, 128), jnp.float32),
            pltpu.SemaphoreType.DMA((NBUF,)),
            pltpu.SemaphoreType.DMA((NBUF,)),
            pltpu.SemaphoreType.DMA,
        ],
    )(functools.partial(_msg_body, split_cols=split_cols, ept=ept))
    return kfn(table, ei_flat)


def _layer2_body(agg_ref, n_ref, w_ref, out_ref):
    nvec = n_ref[...][:, None]
    h1a = jnp.maximum(agg_ref[0, :, :] * nvec, 0.0)
    h1b = jnp.maximum(agg_ref[1, :, :] * nvec, 0.0)
    hw = (jnp.dot(h1a, w_ref[0, :, :], preferred_element_type=jnp.float32)
          + jnp.dot(h1b, w_ref[1, :, :], preferred_element_type=jnp.float32))
    out_ref[...] = hw * nvec


def _layer2(agg1, norm, W2, bm):
    W2r = W2.reshape(2, H1 // 2, H2)
    grid = (NP // bm,)
    return pl.pallas_call(
        _layer2_body,
        grid=grid,
        in_specs=[
            pl.BlockSpec((2, bm, H1 // 2), lambda i: (0, i, 0)),
            pl.BlockSpec((bm,), lambda i: (i,)),
            pl.BlockSpec((2, H1 // 2, H2), lambda i: (0, 0, 0)),
        ],
        out_specs=pl.BlockSpec((bm, H2), lambda i: (i, 0)),
        out_shape=jax.ShapeDtypeStruct((NP, H2), jnp.float32),
    )(agg1, norm, W2r)


def _zfuse_body(agg_ref, n_ref, out_ref):
    z = (agg_ref[0, :, :] + agg_ref[1, :, :]) * n_ref[...][:, None]
    out_ref[...] = z.astype(jnp.bfloat16)


def _zfuse(agg2, norm):
    bm = 2048
    return pl.pallas_call(
        _zfuse_body,
        grid=(NP // bm,),
        in_specs=[
            pl.BlockSpec((2, bm, H2), lambda i: (0, i, 0)),
            pl.BlockSpec((bm,), lambda i: (i,)),
        ],
        out_specs=pl.BlockSpec((bm, H2), lambda i: (i, 0)),
        out_shape=jax.ShapeDtypeStruct((NP, H2), jnp.bfloat16),
    )(agg2, norm)


def _decoder_body(a_ref, b_ref, out_ref):
    res = lax.dot_general(
        a_ref[...], b_ref[...], (((1,), (1,)), ((), ())),
        preferred_element_type=jnp.float32)
    out_ref[...] = res.astype(jnp.bfloat16)


def _decoder(z, bm, bn):
    grid = (pl.cdiv(N, bm), pl.cdiv(N, bn))
    out = pl.pallas_call(
        _decoder_body,
        grid=grid,
        in_specs=[
            pl.BlockSpec((bm, H2), lambda i, j: (i, 0)),
            pl.BlockSpec((bn, H2), lambda i, j: (j, 0)),
        ],
        out_specs=pl.BlockSpec((bm, bn), lambda i, j: (i, j)),
        out_shape=jax.ShapeDtypeStruct((N, N), jnp.bfloat16),
    )(z, z)
    return jnp.reshape(out, (-1,)).astype(jnp.float32)


def kernel(x, edge_index, W1, W2):
    tail = jnp.asarray(N + (np.arange(EPAD - E) % (NP - N)), jnp.int32)
    ei_flat = jnp.concatenate([edge_index[0], tail, edge_index[1], tail])

    deg_p = _deg_partials(ei_flat)
    norm = _norm(deg_p)

    hwn1 = _scaled_matmul_split(x, W1, norm, bm=1024)
    agg1 = _message_pass(hwn1, ei_flat, split_cols=True)

    hwn2 = _layer2(agg1, norm, W2, bm=1024)
    agg2 = _message_pass(hwn2, ei_flat, split_cols=False)

    z = _zfuse(agg2, norm)
    return _decoder(z, bm=2048, bn=2048)
